# baseline (speedup 1.0000x reference)
import functools

import jax
import jax.numpy as jnp
import numpy as np
from jax.experimental import pallas as pl
from jax.experimental.pallas import tpu as pltpu

N_BINS = 15
NB_PAD = 16
R = 512
K = 4


def _ece_kernel(logits_hbm, labels_hbm, out_ref, acc_ref, b0, b1, b2, b3,
                lb0, lb1, lb2, lb3, s0, s1, s2, s3, ls0, ls1, ls2, ls3,
                *, num_samples, n_cols):
    bufs = (b0, b1, b2, b3)
    lbufs = (lb0, lb1, lb2, lb3)
    sems = (s0, s1, s2, s3)
    lsems = (ls0, ls1, ls2, ls3)
    n_chunks = num_samples // R

    def copy(j, t):
        return pltpu.make_async_copy(
            logits_hbm.at[pl.ds(j * R, R), :], bufs[t], sems[t])

    def lcopy(j, t):
        return pltpu.make_async_copy(
            labels_hbm.at[pl.ds(j * R, R), :], lbufs[t], lsems[t])

    for t in range(K):
        copy(t, t).start()
        lcopy(t, t).start()

    acc_ref[...] = jnp.zeros_like(acc_ref)

    def step(jj, t):
        j = jj * K + t
        copy(j, t).wait()
        lcopy(j, t).wait()
        x = bufs[t][...]
        m = jnp.max(x, axis=1, keepdims=True)
        ex = jnp.exp(x - m)
        eqf = (x == m).astype(jnp.float32)
        ones = jnp.ones((n_cols, 1), jnp.float32)
        ci = jax.lax.broadcasted_iota(
            jnp.int32, (n_cols, 2), 0).astype(jnp.float32)
        sel = jax.lax.broadcasted_iota(jnp.int32, (n_cols, 2), 1)
        colv = jnp.where(sel == 0, ci, 1.0)
        s = jax.lax.dot_general(
            ex, ones, (((1,), (0,)), ((), ())),
            precision=jax.lax.Precision.HIGHEST,
            preferred_element_type=jnp.float32)
        pm = jax.lax.dot_general(
            eqf, colv, (((1,), (0,)), ((), ())),
            precision=jax.lax.Precision.HIGHEST,
            preferred_element_type=jnp.float32)
        conf = 1.0 / s
        labf = lbufs[t][...].astype(jnp.float32)
        acc = ((pm[:, 0:1] == labf) & (pm[:, 1:2] == 1.0)).astype(jnp.float32)

        bi = jax.lax.broadcasted_iota(
            jnp.int32, (1, NB_PAD), 1).astype(jnp.float32)
        scale = np.float32(num_samples) / np.float32(N_BINS)
        lo = jnp.round(bi * scale)
        up = jnp.round((bi + 1.0) * scale)
        in_bin = ((conf > lo) & (conf <= up)).astype(jnp.float32)
        cnt = jnp.sum(in_bin, axis=0, keepdims=True)
        sconf = jnp.sum(in_bin * conf, axis=0, keepdims=True)
        sacc = jnp.sum(in_bin * acc, axis=0, keepdims=True)
        acc_ref[...] += jnp.concatenate([cnt, sconf, sacc], axis=0)

        @pl.when(j + K < n_chunks)
        def _refill():
            copy(j + K, t).start()
            lcopy(j + K, t).start()

    def outer(jj, carry):
        for t in range(K):
            step(jj, t)
        return carry

    jax.lax.fori_loop(0, n_chunks // K, outer, jnp.int32(0))

    tot = acc_ref[...]
    cnt_f = tot[0:1, :]
    denom = jnp.maximum(cnt_f, 1.0)
    avg_conf = tot[1:2, :] / denom
    avg_acc = tot[2:3, :] / denom
    prop = cnt_f / np.float32(num_samples)
    out_ref[0] = jnp.sum(jnp.abs(avg_conf - avg_acc) * prop)


@jax.jit
def kernel(logits, labels):
    num_samples, n_cols = logits.shape
    labels2d = labels.astype(jnp.int32).reshape(num_samples, 1)

    body = functools.partial(
        _ece_kernel, num_samples=num_samples, n_cols=n_cols)

    ece = pl.pallas_call(
        body,
        in_specs=[
            pl.BlockSpec(memory_space=pl.ANY),
            pl.BlockSpec(memory_space=pl.ANY),
        ],
        out_specs=pl.BlockSpec(memory_space=pltpu.SMEM),
        out_shape=jax.ShapeDtypeStruct((1,), jnp.float32),
        scratch_shapes=(
            [pltpu.VMEM((3, NB_PAD), jnp.float32)]
            + [pltpu.VMEM((R, n_cols), jnp.float32) for _ in range(K)]
            + [pltpu.VMEM((R, 1), jnp.int32) for _ in range(K)]
            + [pltpu.SemaphoreType.DMA for _ in range(2 * K)]
        ),
    )(logits, labels2d)
    return ece


# speedup vs baseline: 1.9661x; 1.9661x over previous
import functools

import jax
import jax.numpy as jnp
import numpy as np
from jax.experimental import pallas as pl
from jax.experimental.pallas import tpu as pltpu

N_BINS = 15
NB_PAD = 16
R = 512
K = 4


def _ece_kernel(logits_hbm, labels_hbm, out_ref, acc_ref, b0, b1, b2, b3,
                lb0, lb1, lb2, lb3, s0, s1, s2, s3, ls0, ls1, ls2, ls3,
                *, num_samples, n_cols):
    bufs = (b0, b1, b2, b3)
    lbufs = (lb0, lb1, lb2, lb3)
    sems = (s0, s1, s2, s3)
    lsems = (ls0, ls1, ls2, ls3)
    n_chunks = num_samples // R

    def copy(j, t):
        return pltpu.make_async_copy(
            logits_hbm.at[pl.ds(j * R, R), :], bufs[t], sems[t])

    def lcopy(j, t):
        return pltpu.make_async_copy(
            labels_hbm.at[pl.ds(j * R, R), :], lbufs[t], lsems[t])

    for t in range(K):
        copy(t, t).start()
        lcopy(t, t).start()

    acc_ref[...] = jnp.zeros_like(acc_ref)

    def step(jj, t):
        j = jj * K + t
        copy(j, t).wait()
        lcopy(j, t).wait()
        x = bufs[t][...]
        m = jnp.max(x, axis=1, keepdims=True)
        s = jnp.sum(jnp.exp(x - m), axis=1, keepdims=True)
        conf = 1.0 / s
        col = jax.lax.broadcasted_iota(jnp.int32, x.shape, 1)
        pred = jnp.min(jnp.where(x == m, col, n_cols), axis=1, keepdims=True)
        acc = (pred == lbufs[t][...]).astype(jnp.float32)

        bi = jax.lax.broadcasted_iota(
            jnp.int32, (1, NB_PAD), 1).astype(jnp.float32)
        scale = np.float32(num_samples) / np.float32(N_BINS)
        lo = jnp.round(bi * scale)
        up = jnp.round((bi + 1.0) * scale)
        in_bin = ((conf > lo) & (conf <= up)).astype(jnp.float32)
        cnt = jnp.sum(in_bin, axis=0, keepdims=True)
        sconf = jnp.sum(in_bin * conf, axis=0, keepdims=True)
        sacc = jnp.sum(in_bin * acc, axis=0, keepdims=True)
        acc_ref[...] += jnp.concatenate([cnt, sconf, sacc], axis=0)

        @pl.when(j + K < n_chunks)
        def _refill():
            copy(j + K, t).start()
            lcopy(j + K, t).start()

    def outer(jj, carry):
        for t in range(K):
            step(jj, t)
        return carry

    jax.lax.fori_loop(0, n_chunks // K, outer, jnp.int32(0))

    tot = acc_ref[...]
    cnt_f = tot[0:1, :]
    denom = jnp.maximum(cnt_f, 1.0)
    avg_conf = tot[1:2, :] / denom
    avg_acc = tot[2:3, :] / denom
    prop = cnt_f / np.float32(num_samples)
    out_ref[0] = jnp.sum(jnp.abs(avg_conf - avg_acc) * prop)


@jax.jit
def kernel(logits, labels):
    num_samples, n_cols = logits.shape
    labels2d = labels.astype(jnp.int32).reshape(num_samples, 1)

    body = functools.partial(
        _ece_kernel, num_samples=num_samples, n_cols=n_cols)

    ece = pl.pallas_call(
        body,
        in_specs=[
            pl.BlockSpec(memory_space=pl.ANY),
            pl.BlockSpec(memory_space=pl.ANY),
        ],
        out_specs=pl.BlockSpec(memory_space=pltpu.SMEM),
        out_shape=jax.ShapeDtypeStruct((1,), jnp.float32),
        scratch_shapes=(
            [pltpu.VMEM((3, NB_PAD), jnp.float32)]
            + [pltpu.VMEM((R, n_cols), jnp.float32) for _ in range(K)]
            + [pltpu.VMEM((R, 1), jnp.int32) for _ in range(K)]
            + [pltpu.SemaphoreType.DMA for _ in range(2 * K)]
        ),
    )(logits, labels2d)
    return ece
